# Initial kernel scaffold; baseline (speedup 1.0000x reference)
import jax
import jax.numpy as jnp
from jax.experimental import pallas as pl


def kernel(x, edge_index, W_self1, W_neigh1, b1, W_self2, W_neigh2, b2):
    raise NotImplementedError("write your pallas kernel here")


# speedup vs baseline: 5.2099x; 5.2099x over previous
import functools

import jax
import jax.numpy as jnp
from jax import lax
from jax.experimental import pallas as pl
from jax.experimental.pallas import tpu as pltpu
from jax.experimental.pallas import tpu_sc as plsc

N = 10000
E = 320000
D_IN = 128
D_HID = 128
D_OUT = 64

NC = 2
NS = 16
NW = NC * NS
EPW = E // NW
CH = 80
NCHUNK = EPW // CH
NPAD = 10240
RPS = NPAD // NS
DPS = NPAD // NS
ZROWS = 128


@functools.cache
def _make_sc_agg(d, with_deg):
    mesh = plsc.VectorSubcoreMesh(core_axis_name="c", subcore_axis_name="s",
                                  num_cores=NC, num_subcores=NS)
    out_type = [jax.ShapeDtypeStruct((NC, NPAD, d), jnp.float32)]
    scratch = [
        pltpu.VMEM_SHARED((NPAD, d), jnp.float32),
        pltpu.VMEM((CH,), jnp.int32),
        pltpu.VMEM((CH,), jnp.int32),
        pltpu.VMEM((CH, d), jnp.float32),
        pltpu.VMEM((ZROWS, d), jnp.float32),
        pltpu.SemaphoreType.DMA,
    ]
    if with_deg:
        out_type.append(jax.ShapeDtypeStruct((NC, NPAD), jnp.float32))
        scratch += [
            pltpu.VMEM_SHARED((NPAD,), jnp.float32),
            pltpu.VMEM((DPS,), jnp.float32),
            pltpu.VMEM((CH,), jnp.float32),
        ]

    def body(x_hbm, src_hbm, dst_hbm, agg_out, *rest):
        if with_deg:
            (deg_out, agg_sh, src_v, dst_v, rows_v, zb, sem,
             deg_sh, zdeg, ones_v) = rest
        else:
            agg_sh, src_v, dst_v, rows_v, zb, sem = rest
        c = lax.axis_index("c")
        s = lax.axis_index("s")

        def zfill(i, _):
            r = i // (d // 16)
            col = (i % (d // 16)) * 16
            zb[r, pl.ds(col, 16)] = jnp.zeros((16,), jnp.float32)
            return _
        lax.fori_loop(0, ZROWS * (d // 16), zfill, None)
        for j in range(RPS // ZROWS):
            pltpu.sync_copy(zb, agg_sh.at[pl.ds(s * RPS + j * ZROWS, ZROWS)])
        if with_deg:
            def zdfill(i, _):
                zdeg[pl.ds(i * 16, 16)] = jnp.zeros((16,), jnp.float32)
                return _
            lax.fori_loop(0, DPS // 16, zdfill, None)
            pltpu.sync_copy(zdeg, deg_sh.at[pl.ds(s * DPS, DPS)])
            def ofill(i, _):
                ones_v[pl.ds(i * 16, 16)] = jnp.ones((16,), jnp.float32)
                return _
            lax.fori_loop(0, CH // 16, ofill, None)
        plsc.subcore_barrier()

        base = (c * NS + s) * EPW

        def step(i, _):
            off = base + i * CH
            pltpu.sync_copy(src_hbm.at[pl.ds(off, CH)], src_v)
            pltpu.sync_copy(dst_hbm.at[pl.ds(off, CH)], dst_v)
            pltpu.async_copy(x_hbm.at[src_v], rows_v, sem).wait()
            pltpu.sync_copy(rows_v, agg_sh.at[dst_v], add=True)
            if with_deg:
                pltpu.sync_copy(ones_v, deg_sh.at[dst_v], add=True)
            return _
        lax.fori_loop(0, NCHUNK, step, None)

        plsc.subcore_barrier()
        pltpu.sync_copy(agg_sh.at[pl.ds(s * RPS, RPS)],
                        agg_out.at[c, pl.ds(s * RPS, RPS)])
        if with_deg:
            pltpu.sync_copy(deg_sh.at[pl.ds(s * DPS, DPS)],
                            deg_out.at[c, pl.ds(s * DPS, DPS)])

    params = None
    if d % 128 != 0:
        params = pltpu.CompilerParams(use_tc_tiling_on_sc=False)
    return pl.kernel(body, out_type=tuple(out_type), mesh=mesh,
                     scratch_types=scratch, compiler_params=params)


BN = 400
GRID = N // BN


def _dense1_body(x_ref, p0_ref, p1_ref, deg_ref, ws1_ref, wn1_ref, b1_ref,
                 ws2_ref, wn2_ref, y2_ref, z_ref, inv_ref):
    d = jnp.sum(deg_ref[...], axis=0)
    inv = 1.0 / jnp.maximum(d, 1.0)
    inv_ref[...] = inv
    hn = (p0_ref[0] + p1_ref[0]) * inv
    h1 = x_ref[...] @ ws1_ref[...] + hn @ wn1_ref[...] + b1_ref[...]
    h1 = jnp.maximum(h1, 0.0)
    y2_ref[...] = h1 @ wn2_ref[...]
    z_ref[...] = h1 @ ws2_ref[...]


def _dense2_body(z_ref, p0_ref, p1_ref, inv_ref, b2_ref, out_ref):
    logits = z_ref[...] + (p0_ref[0] + p1_ref[0]) * inv_ref[...] + b2_ref[...]
    m = jnp.max(logits, axis=1, keepdims=True)
    t = logits - m
    out_ref[...] = t - jnp.log(jnp.sum(jnp.exp(t), axis=1, keepdims=True))


def kernel(x, edge_index, W_self1, W_neigh1, b1, W_self2, W_neigh2, b2):
    src = edge_index[0]
    dst = edge_index[1]

    agg1p, degp = _make_sc_agg(D_IN, True)(x, src, dst)
    degp = degp.reshape(NC, NPAD, 1)

    y2, z, inv_deg = pl.pallas_call(
        _dense1_body,
        grid=(GRID,),
        in_specs=[
            pl.BlockSpec((BN, D_IN), lambda i: (i, 0)),
            pl.BlockSpec((1, BN, D_IN), lambda i: (0, i, 0)),
            pl.BlockSpec((1, BN, D_IN), lambda i: (1, i, 0)),
            pl.BlockSpec((NC, BN, 1), lambda i: (0, i, 0)),
            pl.BlockSpec((D_IN, D_HID), lambda i: (0, 0)),
            pl.BlockSpec((D_IN, D_HID), lambda i: (0, 0)),
            pl.BlockSpec((1, D_HID), lambda i: (0, 0)),
            pl.BlockSpec((D_HID, D_OUT), lambda i: (0, 0)),
            pl.BlockSpec((D_HID, D_OUT), lambda i: (0, 0)),
        ],
        out_specs=[
            pl.BlockSpec((BN, D_OUT), lambda i: (i, 0)),
            pl.BlockSpec((BN, D_OUT), lambda i: (i, 0)),
            pl.BlockSpec((BN, 1), lambda i: (i, 0)),
        ],
        out_shape=[
            jax.ShapeDtypeStruct((N, D_OUT), jnp.float32),
            jax.ShapeDtypeStruct((N, D_OUT), jnp.float32),
            jax.ShapeDtypeStruct((N, 1), jnp.float32),
        ],
    )(x, agg1p, agg1p, degp, W_self1, W_neigh1, b1.reshape(1, D_HID),
      W_self2, W_neigh2)

    (agg2p,) = _make_sc_agg(D_OUT, False)(y2, src, dst)

    out = pl.pallas_call(
        _dense2_body,
        grid=(GRID,),
        in_specs=[
            pl.BlockSpec((BN, D_OUT), lambda i: (i, 0)),
            pl.BlockSpec((1, BN, D_OUT), lambda i: (0, i, 0)),
            pl.BlockSpec((1, BN, D_OUT), lambda i: (1, i, 0)),
            pl.BlockSpec((BN, 1), lambda i: (i, 0)),
            pl.BlockSpec((1, D_OUT), lambda i: (0, 0)),
        ],
        out_specs=pl.BlockSpec((BN, D_OUT), lambda i: (i, 0)),
        out_shape=jax.ShapeDtypeStruct((N, D_OUT), jnp.float32),
    )(z, agg2p, agg2p, inv_deg, b2.reshape(1, D_OUT))

    return out


# speedup vs baseline: 9.5696x; 1.8368x over previous
import functools

import jax
import jax.numpy as jnp
from jax import lax
from jax.experimental import pallas as pl
from jax.experimental.pallas import tpu as pltpu
from jax.experimental.pallas import tpu_sc as plsc

N = 10000
E = 320000
D_IN = 128
D_HID = 128
D_OUT = 64

NC = 2
NS = 16
NW = NC * NS
EPW = E // NW
K = 5
NPAD = 10240
RPS = NPAD // NS
DPS = NPAD // NS


@functools.cache
def _make_sc_agg(d, with_deg):
    mesh = plsc.VectorSubcoreMesh(core_axis_name="c", subcore_axis_name="s",
                                  num_cores=NC, num_subcores=NS)
    CH = 40 if d > 64 else 80
    NCHUNK = EPW // CH
    out_type = [jax.ShapeDtypeStruct((NC, NPAD, d), jnp.float32)]
    scratch = (
        [pltpu.VMEM_SHARED((NPAD, d), jnp.float32)]
        + [pltpu.VMEM((CH,), jnp.int32) for _ in range(K)]
        + [pltpu.VMEM((CH,), jnp.int32) for _ in range(K)]
        + [pltpu.VMEM((CH, d), jnp.float32) for _ in range(K)]
        + [pltpu.SemaphoreType.DMA]
        + [pltpu.SemaphoreType.DMA for _ in range(K)]
        + [pltpu.SemaphoreType.DMA for _ in range(K)]
    )
    if with_deg:
        out_type.append(jax.ShapeDtypeStruct((NC, NPAD), jnp.float32))
        scratch += [
            pltpu.VMEM_SHARED((NPAD,), jnp.float32),
            pltpu.VMEM((DPS,), jnp.float32),
            pltpu.VMEM((CH,), jnp.float32),
        ]

    def body(x_hbm, src_hbm, dst_hbm, agg_out, *rest):
        rest = list(rest)
        deg_out = rest.pop(0) if with_deg else None
        agg_sh = rest.pop(0)
        src_v = [rest.pop(0) for _ in range(K)]
        dst_v = [rest.pop(0) for _ in range(K)]
        rows_v = [rest.pop(0) for _ in range(K)]
        semi = rest.pop(0)
        semg = [rest.pop(0) for _ in range(K)]
        sems = [rest.pop(0) for _ in range(K)]
        if with_deg:
            deg_sh, zdeg, ones_v = rest
        c = lax.axis_index("c")
        s = lax.axis_index("s")

        def zfill(i, _):
            r = i // (d // 16)
            col = (i % (d // 16)) * 16
            rows_v[0][r, pl.ds(col, 16)] = jnp.zeros((16,), jnp.float32)
            return _
        lax.fori_loop(0, CH * (d // 16), zfill, None)
        for j in range(RPS // CH):
            pltpu.sync_copy(rows_v[0], agg_sh.at[pl.ds(s * RPS + j * CH, CH)])
        if with_deg:
            def zdfill(i, _):
                zdeg[pl.ds(i * 16, 16)] = jnp.zeros((16,), jnp.float32)
                return _
            lax.fori_loop(0, DPS // 16, zdfill, None)
            pltpu.sync_copy(zdeg, deg_sh.at[pl.ds(s * DPS, DPS)])
            offs = list(range(0, CH - 15, 16))
            if CH % 16:
                offs.append(CH - 16)
            for o in offs:
                ones_v[pl.ds(o, 16)] = jnp.ones((16,), jnp.float32)
        plsc.subcore_barrier()

        base = (c * NS + s) * EPW

        def group(g, _):
            off0 = base + g * (K * CH)
            idx_descs = []
            for k in range(K):
                off = off0 + k * CH
                idx_descs.append(pltpu.async_copy(
                    src_hbm.at[pl.ds(off, CH)], src_v[k], semi))
                idx_descs.append(pltpu.async_copy(
                    dst_hbm.at[pl.ds(off, CH)], dst_v[k], semi))
            for dsc in idx_descs:
                dsc.wait()
            g_descs = [pltpu.async_copy(x_hbm.at[src_v[k]], rows_v[k], semg[k])
                       for k in range(K)]
            s_descs = []
            for k in range(K):
                g_descs[k].wait()
                s_descs.append(pltpu.async_copy(
                    rows_v[k], agg_sh.at[dst_v[k]], sems[k], add=True))
                if with_deg:
                    s_descs.append(pltpu.async_copy(
                        ones_v, deg_sh.at[dst_v[k]], sems[k], add=True))
            for dsc in s_descs:
                dsc.wait()
            return _
        assert NCHUNK % K == 0
        lax.fori_loop(0, NCHUNK // K, group, None)

        plsc.subcore_barrier()
        pltpu.sync_copy(agg_sh.at[pl.ds(s * RPS, RPS)],
                        agg_out.at[c, pl.ds(s * RPS, RPS)])
        if with_deg:
            pltpu.sync_copy(deg_sh.at[pl.ds(s * DPS, DPS)],
                            deg_out.at[c, pl.ds(s * DPS, DPS)])

    params = None
    if d % 128 != 0:
        params = pltpu.CompilerParams(use_tc_tiling_on_sc=False)
    return pl.kernel(body, out_type=tuple(out_type), mesh=mesh,
                     scratch_types=scratch, compiler_params=params)


BN = 400
GRID = N // BN


def _dense1_body(x_ref, p0_ref, p1_ref, deg_ref, ws1_ref, wn1_ref, b1_ref,
                 ws2_ref, wn2_ref, y2_ref, z_ref, inv_ref):
    d = jnp.sum(deg_ref[...], axis=0)
    inv = 1.0 / jnp.maximum(d, 1.0)
    inv_ref[...] = inv
    hn = (p0_ref[0] + p1_ref[0]) * inv
    h1 = x_ref[...] @ ws1_ref[...] + hn @ wn1_ref[...] + b1_ref[...]
    h1 = jnp.maximum(h1, 0.0)
    y2_ref[...] = h1 @ wn2_ref[...]
    z_ref[...] = h1 @ ws2_ref[...]


def _dense2_body(z_ref, p0_ref, p1_ref, inv_ref, b2_ref, out_ref):
    logits = z_ref[...] + (p0_ref[0] + p1_ref[0]) * inv_ref[...] + b2_ref[...]
    m = jnp.max(logits, axis=1, keepdims=True)
    t = logits - m
    out_ref[...] = t - jnp.log(jnp.sum(jnp.exp(t), axis=1, keepdims=True))


def kernel(x, edge_index, W_self1, W_neigh1, b1, W_self2, W_neigh2, b2):
    src = edge_index[0]
    dst = edge_index[1]

    agg1p, degp = _make_sc_agg(D_IN, True)(x, src, dst)
    degp = degp.reshape(NC, NPAD, 1)

    y2, z, inv_deg = pl.pallas_call(
        _dense1_body,
        grid=(GRID,),
        in_specs=[
            pl.BlockSpec((BN, D_IN), lambda i: (i, 0)),
            pl.BlockSpec((1, BN, D_IN), lambda i: (0, i, 0)),
            pl.BlockSpec((1, BN, D_IN), lambda i: (1, i, 0)),
            pl.BlockSpec((NC, BN, 1), lambda i: (0, i, 0)),
            pl.BlockSpec((D_IN, D_HID), lambda i: (0, 0)),
            pl.BlockSpec((D_IN, D_HID), lambda i: (0, 0)),
            pl.BlockSpec((1, D_HID), lambda i: (0, 0)),
            pl.BlockSpec((D_HID, D_OUT), lambda i: (0, 0)),
            pl.BlockSpec((D_HID, D_OUT), lambda i: (0, 0)),
        ],
        out_specs=[
            pl.BlockSpec((BN, D_OUT), lambda i: (i, 0)),
            pl.BlockSpec((BN, D_OUT), lambda i: (i, 0)),
            pl.BlockSpec((BN, 1), lambda i: (i, 0)),
        ],
        out_shape=[
            jax.ShapeDtypeStruct((N, D_OUT), jnp.float32),
            jax.ShapeDtypeStruct((N, D_OUT), jnp.float32),
            jax.ShapeDtypeStruct((N, 1), jnp.float32),
        ],
    )(x, agg1p, agg1p, degp, W_self1, W_neigh1, b1.reshape(1, D_HID),
      W_self2, W_neigh2)

    (agg2p,) = _make_sc_agg(D_OUT, False)(y2, src, dst)

    out = pl.pallas_call(
        _dense2_body,
        grid=(GRID,),
        in_specs=[
            pl.BlockSpec((BN, D_OUT), lambda i: (i, 0)),
            pl.BlockSpec((1, BN, D_OUT), lambda i: (0, i, 0)),
            pl.BlockSpec((1, BN, D_OUT), lambda i: (1, i, 0)),
            pl.BlockSpec((BN, 1), lambda i: (i, 0)),
            pl.BlockSpec((1, D_OUT), lambda i: (0, 0)),
        ],
        out_specs=pl.BlockSpec((BN, D_OUT), lambda i: (i, 0)),
        out_shape=jax.ShapeDtypeStruct((N, D_OUT), jnp.float32),
    )(z, agg2p, agg2p, inv_deg, b2.reshape(1, D_OUT))

    return out


# speedup vs baseline: 11.3818x; 1.1894x over previous
import functools

import jax
import jax.numpy as jnp
from jax import lax
from jax.experimental import pallas as pl
from jax.experimental.pallas import tpu as pltpu
from jax.experimental.pallas import tpu_sc as plsc

N = 10000
E = 320000
D_IN = 128
D_HID = 128
D_OUT = 64

NC = 2
NS = 16
NW = NC * NS
EPW = E // NW
K = 5
NPAD = 10240
RPS = NPAD // NS
DPS = NPAD // NS


@functools.cache
def _make_sc_agg(d, with_deg):
    mesh = plsc.VectorSubcoreMesh(core_axis_name="c", subcore_axis_name="s",
                                  num_cores=NC, num_subcores=NS)
    CH = 40 if d > 64 else 80
    NCHUNK = EPW // CH
    out_type = [jax.ShapeDtypeStruct((NC, NPAD, d), jnp.float32)]
    scratch = (
        [pltpu.VMEM_SHARED((NPAD, d), jnp.float32)]
        + [pltpu.VMEM((CH,), jnp.int32) for _ in range(K)]
        + [pltpu.VMEM((CH,), jnp.int32) for _ in range(K)]
        + [pltpu.VMEM((CH, d), jnp.float32) for _ in range(K)]
        + [pltpu.SemaphoreType.DMA for _ in range(K)]
        + [pltpu.SemaphoreType.DMA for _ in range(K)]
        + [pltpu.SemaphoreType.DMA for _ in range(K)]
    )
    if with_deg:
        out_type.append(jax.ShapeDtypeStruct((NC, NPAD), jnp.float32))
        scratch += [
            pltpu.VMEM_SHARED((NPAD,), jnp.float32),
            pltpu.VMEM((DPS,), jnp.float32),
            pltpu.VMEM((CH,), jnp.float32),
        ]

    def body(x_hbm, src_hbm, dst_hbm, agg_out, *rest):
        rest = list(rest)
        deg_out = rest.pop(0) if with_deg else None
        agg_sh = rest.pop(0)
        src_v = [rest.pop(0) for _ in range(K)]
        dst_v = [rest.pop(0) for _ in range(K)]
        rows_v = [rest.pop(0) for _ in range(K)]
        semi = [rest.pop(0) for _ in range(K)]
        semg = [rest.pop(0) for _ in range(K)]
        sems = [rest.pop(0) for _ in range(K)]
        if with_deg:
            deg_sh, zdeg, ones_v = rest
        c = lax.axis_index("c")
        s = lax.axis_index("s")

        def zfill(i, _):
            r = i // (d // 16)
            col = (i % (d // 16)) * 16
            rows_v[0][r, pl.ds(col, 16)] = jnp.zeros((16,), jnp.float32)
            return _
        lax.fori_loop(0, CH * (d // 16), zfill, None)
        for j in range(RPS // CH):
            pltpu.sync_copy(rows_v[0], agg_sh.at[pl.ds(s * RPS + j * CH, CH)])
        if with_deg:
            def zdfill(i, _):
                zdeg[pl.ds(i * 16, 16)] = jnp.zeros((16,), jnp.float32)
                return _
            lax.fori_loop(0, DPS // 16, zdfill, None)
            pltpu.sync_copy(zdeg, deg_sh.at[pl.ds(s * DPS, DPS)])
            offs = list(range(0, CH - 15, 16))
            if CH % 16:
                offs.append(CH - 16)
            for o in offs:
                ones_v[pl.ds(o, 16)] = jnp.ones((16,), jnp.float32)
        plsc.subcore_barrier()

        base = (c * NS + s) * EPW

        def issue_idx(i, sl):
            pltpu.async_copy(src_hbm.at[pl.ds(base + i * CH, CH)],
                             src_v[sl], semi[sl])
            pltpu.async_copy(dst_hbm.at[pl.ds(base + i * CH, CH)],
                             dst_v[sl], semi[sl])

        def wait_idx(sl):
            for ref in (src_v[sl], dst_v[sl]):
                pltpu.make_async_copy(src_hbm.at[pl.ds(0, CH)], ref,
                                      semi[sl]).wait()

        def issue_gather(sl):
            pltpu.async_copy(x_hbm.at[src_v[sl]], rows_v[sl], semg[sl])

        def wait_gather(sl):
            pltpu.make_async_copy(x_hbm.at[src_v[sl]], rows_v[sl],
                                  semg[sl]).wait()

        def issue_scatter(sl):
            pltpu.async_copy(rows_v[sl], agg_sh.at[dst_v[sl]], sems[sl],
                             add=True)
            if with_deg:
                pltpu.async_copy(ones_v, deg_sh.at[dst_v[sl]], sems[sl],
                                 add=True)

        def wait_scatter(sl):
            pltpu.make_async_copy(rows_v[sl], agg_sh.at[dst_v[sl]],
                                  sems[sl]).wait()
            if with_deg:
                pltpu.make_async_copy(ones_v, deg_sh.at[dst_v[sl]],
                                      sems[sl]).wait()

        def chunk_body(i, j, drain, nxt2, nxt1):
            if drain:
                wait_scatter((j + 2) % K)
            if nxt2:
                issue_idx(i + 2, (j + 2) % K)
            if nxt1:
                wait_idx((j + 1) % K)
                issue_gather((j + 1) % K)
            wait_gather(j)
            issue_scatter(j)

        assert NCHUNK % K == 0 and NCHUNK // K >= 2
        G = NCHUNK // K
        issue_idx(0, 0)
        issue_idx(1, 1)
        wait_idx(0)
        issue_gather(0)
        for j in range(K):
            chunk_body(j, j, j + 2 >= K, True, True)

        def steady(t, _):
            for j in range(K):
                chunk_body(t * K + j, j, True, True, True)
            return _
        lax.fori_loop(1, G - 1, steady, None)

        for j in range(K):
            i = (G - 1) * K + j
            chunk_body(i, j, True, i + 2 < NCHUNK, i + 1 < NCHUNK)
        for sl in (K - 3, K - 2, K - 1):
            wait_scatter(sl)

        plsc.subcore_barrier()
        pltpu.sync_copy(agg_sh.at[pl.ds(s * RPS, RPS)],
                        agg_out.at[c, pl.ds(s * RPS, RPS)])
        if with_deg:
            pltpu.sync_copy(deg_sh.at[pl.ds(s * DPS, DPS)],
                            deg_out.at[c, pl.ds(s * DPS, DPS)])

    params = None
    if d % 128 != 0:
        params = pltpu.CompilerParams(use_tc_tiling_on_sc=False)
    return pl.kernel(body, out_type=tuple(out_type), mesh=mesh,
                     scratch_types=scratch, compiler_params=params)


BN = 400
GRID = N // BN


def _dense1_body(x_ref, p0_ref, p1_ref, deg_ref, ws1_ref, wn1_ref, b1_ref,
                 ws2_ref, wn2_ref, y2_ref, z_ref, inv_ref):
    d = jnp.sum(deg_ref[...], axis=0)
    inv = 1.0 / jnp.maximum(d, 1.0)
    inv_ref[...] = inv
    hn = (p0_ref[0] + p1_ref[0]) * inv
    h1 = x_ref[...] @ ws1_ref[...] + hn @ wn1_ref[...] + b1_ref[...]
    h1 = jnp.maximum(h1, 0.0)
    y2_ref[...] = h1 @ wn2_ref[...]
    z_ref[...] = h1 @ ws2_ref[...]


def _dense2_body(z_ref, p0_ref, p1_ref, inv_ref, b2_ref, out_ref):
    logits = z_ref[...] + (p0_ref[0] + p1_ref[0]) * inv_ref[...] + b2_ref[...]
    m = jnp.max(logits, axis=1, keepdims=True)
    t = logits - m
    out_ref[...] = t - jnp.log(jnp.sum(jnp.exp(t), axis=1, keepdims=True))


def kernel(x, edge_index, W_self1, W_neigh1, b1, W_self2, W_neigh2, b2):
    src = edge_index[0]
    dst = edge_index[1]

    agg1p, degp = _make_sc_agg(D_IN, True)(x, src, dst)
    degp = degp.reshape(NC, NPAD, 1)

    y2, z, inv_deg = pl.pallas_call(
        _dense1_body,
        grid=(GRID,),
        in_specs=[
            pl.BlockSpec((BN, D_IN), lambda i: (i, 0)),
            pl.BlockSpec((1, BN, D_IN), lambda i: (0, i, 0)),
            pl.BlockSpec((1, BN, D_IN), lambda i: (1, i, 0)),
            pl.BlockSpec((NC, BN, 1), lambda i: (0, i, 0)),
            pl.BlockSpec((D_IN, D_HID), lambda i: (0, 0)),
            pl.BlockSpec((D_IN, D_HID), lambda i: (0, 0)),
            pl.BlockSpec((1, D_HID), lambda i: (0, 0)),
            pl.BlockSpec((D_HID, D_OUT), lambda i: (0, 0)),
            pl.BlockSpec((D_HID, D_OUT), lambda i: (0, 0)),
        ],
        out_specs=[
            pl.BlockSpec((BN, D_OUT), lambda i: (i, 0)),
            pl.BlockSpec((BN, D_OUT), lambda i: (i, 0)),
            pl.BlockSpec((BN, 1), lambda i: (i, 0)),
        ],
        out_shape=[
            jax.ShapeDtypeStruct((N, D_OUT), jnp.float32),
            jax.ShapeDtypeStruct((N, D_OUT), jnp.float32),
            jax.ShapeDtypeStruct((N, 1), jnp.float32),
        ],
    )(x, agg1p, agg1p, degp, W_self1, W_neigh1, b1.reshape(1, D_HID),
      W_self2, W_neigh2)

    (agg2p,) = _make_sc_agg(D_OUT, False)(y2, src, dst)

    out = pl.pallas_call(
        _dense2_body,
        grid=(GRID,),
        in_specs=[
            pl.BlockSpec((BN, D_OUT), lambda i: (i, 0)),
            pl.BlockSpec((1, BN, D_OUT), lambda i: (0, i, 0)),
            pl.BlockSpec((1, BN, D_OUT), lambda i: (1, i, 0)),
            pl.BlockSpec((BN, 1), lambda i: (i, 0)),
            pl.BlockSpec((1, D_OUT), lambda i: (0, 0)),
        ],
        out_specs=pl.BlockSpec((BN, D_OUT), lambda i: (i, 0)),
        out_shape=jax.ShapeDtypeStruct((N, D_OUT), jnp.float32),
    )(z, agg2p, agg2p, inv_deg, b2.reshape(1, D_OUT))

    return out


# speedup vs baseline: 12.1918x; 1.0712x over previous
import functools

import jax
import jax.numpy as jnp
from jax import lax
from jax.experimental import pallas as pl
from jax.experimental.pallas import tpu as pltpu
from jax.experimental.pallas import tpu_sc as plsc

N = 10000
E = 320000
D_IN = 128
D_HID = 128
D_OUT = 64

NC = 2
NS = 16
NW = NC * NS
EPW = E // NW
K = 5
NPAD = 10240
RPS = NPAD // NS
DPS = NPAD // NS


@functools.cache
def _make_sc_agg(d, with_deg):
    mesh = plsc.VectorSubcoreMesh(core_axis_name="c", subcore_axis_name="s",
                                  num_cores=NC, num_subcores=NS)
    CH = 40 if d > 64 else 80
    NCHUNK = EPW // CH
    out_type = [jax.ShapeDtypeStruct((NC, NPAD, d), jnp.float32)]
    scratch = (
        [pltpu.VMEM_SHARED((NPAD, d), jnp.float32)]
        + [pltpu.VMEM((CH,), jnp.int32) for _ in range(K)]
        + [pltpu.VMEM((CH,), jnp.int32) for _ in range(K)]
        + [pltpu.VMEM((CH, d), jnp.float32) for _ in range(K)]
        + [pltpu.SemaphoreType.DMA for _ in range(K)]
        + [pltpu.SemaphoreType.DMA for _ in range(K)]
        + [pltpu.SemaphoreType.DMA for _ in range(K)]
    )
    if with_deg:
        out_type.append(jax.ShapeDtypeStruct((NC, NPAD), jnp.float32))
        scratch += [
            pltpu.VMEM_SHARED((NPAD,), jnp.float32),
            pltpu.VMEM((DPS,), jnp.float32),
            pltpu.VMEM((CH,), jnp.float32),
        ]

    def body(x_hbm, src_hbm, dst_hbm, agg_out, *rest):
        rest = list(rest)
        deg_out = rest.pop(0) if with_deg else None
        agg_sh = rest.pop(0)
        src_v = [rest.pop(0) for _ in range(K)]
        dst_v = [rest.pop(0) for _ in range(K)]
        rows_v = [rest.pop(0) for _ in range(K)]
        semi = [rest.pop(0) for _ in range(K)]
        semg = [rest.pop(0) for _ in range(K)]
        sems = [rest.pop(0) for _ in range(K)]
        if with_deg:
            deg_sh, zdeg, ones_v = rest
        c = lax.axis_index("c")
        s = lax.axis_index("s")

        def zfill(i, _):
            r = i // (d // 16)
            col = (i % (d // 16)) * 16
            rows_v[0][r, pl.ds(col, 16)] = jnp.zeros((16,), jnp.float32)
            return _
        lax.fori_loop(0, CH * (d // 16), zfill, None)
        for j in range(RPS // CH):
            pltpu.sync_copy(rows_v[0], agg_sh.at[pl.ds(s * RPS + j * CH, CH)])
        if with_deg:
            def zdfill(i, _):
                zdeg[pl.ds(i * 16, 16)] = jnp.zeros((16,), jnp.float32)
                return _
            lax.fori_loop(0, DPS // 16, zdfill, None)
            pltpu.sync_copy(zdeg, deg_sh.at[pl.ds(s * DPS, DPS)])
            offs = list(range(0, CH - 15, 16))
            if CH % 16:
                offs.append(CH - 16)
            for o in offs:
                ones_v[pl.ds(o, 16)] = jnp.ones((16,), jnp.float32)
        plsc.subcore_barrier()

        base = (c * NS + s) * EPW

        def issue_idx(i, sl):
            pltpu.async_copy(src_hbm.at[pl.ds(base + i * CH, CH)],
                             src_v[sl], semi[sl])
            pltpu.async_copy(dst_hbm.at[pl.ds(base + i * CH, CH)],
                             dst_v[sl], semi[sl])

        def wait_idx(sl):
            for ref in (src_v[sl], dst_v[sl]):
                pltpu.make_async_copy(src_hbm.at[pl.ds(0, CH)], ref,
                                      semi[sl]).wait()

        def issue_gather(sl):
            pltpu.async_copy(x_hbm.at[src_v[sl]], rows_v[sl], semg[sl])

        def wait_gather(sl):
            pltpu.make_async_copy(x_hbm.at[src_v[sl]], rows_v[sl],
                                  semg[sl]).wait()

        def issue_scatter(sl):
            pltpu.async_copy(rows_v[sl], agg_sh.at[dst_v[sl]], sems[sl],
                             add=True)
            if with_deg:
                pltpu.async_copy(ones_v, deg_sh.at[dst_v[sl]], sems[sl],
                                 add=True)

        def wait_scatter(sl):
            pltpu.make_async_copy(rows_v[sl], agg_sh.at[dst_v[sl]],
                                  sems[sl]).wait()
            if with_deg:
                pltpu.make_async_copy(ones_v, deg_sh.at[dst_v[sl]],
                                      sems[sl]).wait()

        def chunk_body(i, j, drain, nxt2, nxt1):
            if drain:
                wait_scatter((j + 2) % K)
            if nxt2:
                issue_idx(i + 2, (j + 2) % K)
            if nxt1:
                wait_idx((j + 1) % K)
                issue_gather((j + 1) % K)
            wait_gather(j)
            issue_scatter(j)

        assert NCHUNK % K == 0 and NCHUNK // K >= 2
        G = NCHUNK // K
        issue_idx(0, 0)
        issue_idx(1, 1)
        wait_idx(0)
        issue_gather(0)
        for j in range(K):
            chunk_body(j, j, j + 2 >= K, True, True)

        def steady(t, _):
            for j in range(K):
                chunk_body(t * K + j, j, True, True, True)
            return _
        lax.fori_loop(1, G - 1, steady, None)

        for j in range(K):
            i = (G - 1) * K + j
            chunk_body(i, j, True, i + 2 < NCHUNK, i + 1 < NCHUNK)
        for sl in (K - 3, K - 2, K - 1):
            wait_scatter(sl)

        plsc.subcore_barrier()
        pltpu.sync_copy(agg_sh.at[pl.ds(s * RPS, RPS)],
                        agg_out.at[c, pl.ds(s * RPS, RPS)])
        if with_deg:
            pltpu.sync_copy(deg_sh.at[pl.ds(s * DPS, DPS)],
                            deg_out.at[c, pl.ds(s * DPS, DPS)])

    params = None
    if d % 128 != 0:
        params = pltpu.CompilerParams(use_tc_tiling_on_sc=False)
    return pl.kernel(body, out_type=tuple(out_type), mesh=mesh,
                     scratch_types=scratch, compiler_params=params)


BN = 2000
GRID = N // BN


def _self1_body(x_ref, ws1_ref, b1_ref, xs_ref):
    xs_ref[...] = x_ref[...] @ ws1_ref[...] + b1_ref[...]


def _dense1_body(xs_ref, p0_ref, p1_ref, deg_ref, wn1_ref,
                 ws2_ref, wn2_ref, y2_ref, z_ref, inv_ref):
    d = jnp.sum(deg_ref[...], axis=0)
    inv = 1.0 / jnp.maximum(d, 1.0)
    inv_ref[...] = inv
    hn = (p0_ref[0] + p1_ref[0]) * inv
    h1 = jnp.maximum(xs_ref[...] + hn @ wn1_ref[...], 0.0)
    y2_ref[...] = h1 @ wn2_ref[...]
    z_ref[...] = h1 @ ws2_ref[...]


def _dense2_body(z_ref, p0_ref, p1_ref, inv_ref, b2_ref, out_ref):
    logits = z_ref[...] + (p0_ref[0] + p1_ref[0]) * inv_ref[...] + b2_ref[...]
    m = jnp.max(logits, axis=1, keepdims=True)
    t = logits - m
    out_ref[...] = t - jnp.log(jnp.sum(jnp.exp(t), axis=1, keepdims=True))


def kernel(x, edge_index, W_self1, W_neigh1, b1, W_self2, W_neigh2, b2):
    src = edge_index[0]
    dst = edge_index[1]

    agg1p, degp = _make_sc_agg(D_IN, True)(x, src, dst)
    degp = degp.reshape(NC, NPAD, 1)

    xs = pl.pallas_call(
        _self1_body,
        grid=(GRID,),
        in_specs=[
            pl.BlockSpec((BN, D_IN), lambda i: (i, 0)),
            pl.BlockSpec((D_IN, D_HID), lambda i: (0, 0)),
            pl.BlockSpec((1, D_HID), lambda i: (0, 0)),
        ],
        out_specs=pl.BlockSpec((BN, D_HID), lambda i: (i, 0)),
        out_shape=jax.ShapeDtypeStruct((N, D_HID), jnp.float32),
    )(x, W_self1, b1.reshape(1, D_HID))

    y2, z, inv_deg = pl.pallas_call(
        _dense1_body,
        grid=(GRID,),
        in_specs=[
            pl.BlockSpec((BN, D_HID), lambda i: (i, 0)),
            pl.BlockSpec((1, BN, D_IN), lambda i: (0, i, 0)),
            pl.BlockSpec((1, BN, D_IN), lambda i: (1, i, 0)),
            pl.BlockSpec((NC, BN, 1), lambda i: (0, i, 0)),
            pl.BlockSpec((D_IN, D_HID), lambda i: (0, 0)),
            pl.BlockSpec((D_HID, D_OUT), lambda i: (0, 0)),
            pl.BlockSpec((D_HID, D_OUT), lambda i: (0, 0)),
        ],
        out_specs=[
            pl.BlockSpec((BN, D_OUT), lambda i: (i, 0)),
            pl.BlockSpec((BN, D_OUT), lambda i: (i, 0)),
            pl.BlockSpec((BN, 1), lambda i: (i, 0)),
        ],
        out_shape=[
            jax.ShapeDtypeStruct((N, D_OUT), jnp.float32),
            jax.ShapeDtypeStruct((N, D_OUT), jnp.float32),
            jax.ShapeDtypeStruct((N, 1), jnp.float32),
        ],
    )(xs, agg1p, agg1p, degp, W_neigh1, W_self2, W_neigh2)

    (agg2p,) = _make_sc_agg(D_OUT, False)(y2, src, dst)

    out = pl.pallas_call(
        _dense2_body,
        grid=(GRID,),
        in_specs=[
            pl.BlockSpec((BN, D_OUT), lambda i: (i, 0)),
            pl.BlockSpec((1, BN, D_OUT), lambda i: (0, i, 0)),
            pl.BlockSpec((1, BN, D_OUT), lambda i: (1, i, 0)),
            pl.BlockSpec((BN, 1), lambda i: (i, 0)),
            pl.BlockSpec((1, D_OUT), lambda i: (0, 0)),
        ],
        out_specs=pl.BlockSpec((BN, D_OUT), lambda i: (i, 0)),
        out_shape=jax.ShapeDtypeStruct((N, D_OUT), jnp.float32),
    )(z, agg2p, agg2p, inv_deg, b2.reshape(1, D_OUT))

    return out


# speedup vs baseline: 12.9782x; 1.0645x over previous
import functools

import jax
import jax.numpy as jnp
from jax import lax
from jax.experimental import pallas as pl
from jax.experimental.pallas import tpu as pltpu
from jax.experimental.pallas import tpu_sc as plsc

N = 10000
E = 320000
D_IN = 128
D_HID = 128
D_OUT = 64

NC = 2
NS = 16
NW = NC * NS
EPW = E // NW
K = 5
NPAD = 10240
RPS = NPAD // NS
DPS = NPAD // NS


@functools.cache
def _make_sc_agg(d, with_deg):
    mesh = plsc.VectorSubcoreMesh(core_axis_name="c", subcore_axis_name="s",
                                  num_cores=NC, num_subcores=NS)
    CH = 40 if d > 64 else 200
    NCHUNK = EPW // CH
    out_type = [jax.ShapeDtypeStruct((NC, NPAD, d), jnp.float32)]
    scratch = (
        [pltpu.VMEM_SHARED((NPAD, d), jnp.float32)]
        + [pltpu.VMEM((CH,), jnp.int32) for _ in range(K)]
        + [pltpu.VMEM((CH,), jnp.int32) for _ in range(K)]
        + [pltpu.VMEM((CH, d), jnp.float32) for _ in range(K)]
        + [pltpu.SemaphoreType.DMA for _ in range(K)]
        + [pltpu.SemaphoreType.DMA for _ in range(K)]
        + [pltpu.SemaphoreType.DMA for _ in range(K)]
    )
    if with_deg:
        out_type.append(jax.ShapeDtypeStruct((NC, NPAD), jnp.float32))
        scratch += [
            pltpu.VMEM_SHARED((NPAD,), jnp.float32),
            pltpu.VMEM((DPS,), jnp.float32),
            pltpu.VMEM((CH,), jnp.float32),
        ]

    def body(x_hbm, src_hbm, dst_hbm, agg_out, *rest):
        rest = list(rest)
        deg_out = rest.pop(0) if with_deg else None
        agg_sh = rest.pop(0)
        src_v = [rest.pop(0) for _ in range(K)]
        dst_v = [rest.pop(0) for _ in range(K)]
        rows_v = [rest.pop(0) for _ in range(K)]
        semi = [rest.pop(0) for _ in range(K)]
        semg = [rest.pop(0) for _ in range(K)]
        sems = [rest.pop(0) for _ in range(K)]
        if with_deg:
            deg_sh, zdeg, ones_v = rest
        c = lax.axis_index("c")
        s = lax.axis_index("s")

        ZR = 40
        def zfill(i, _):
            r = i // (d // 16)
            col = (i % (d // 16)) * 16
            rows_v[0][r, pl.ds(col, 16)] = jnp.zeros((16,), jnp.float32)
            return _
        lax.fori_loop(0, ZR * (d // 16), zfill, None)
        for j in range(RPS // ZR):
            pltpu.sync_copy(rows_v[0].at[pl.ds(0, ZR)],
                            agg_sh.at[pl.ds(s * RPS + j * ZR, ZR)])
        if with_deg:
            def zdfill(i, _):
                zdeg[pl.ds(i * 16, 16)] = jnp.zeros((16,), jnp.float32)
                return _
            lax.fori_loop(0, DPS // 16, zdfill, None)
            pltpu.sync_copy(zdeg, deg_sh.at[pl.ds(s * DPS, DPS)])
            offs = list(range(0, CH - 15, 16))
            if CH % 16:
                offs.append(CH - 16)
            for o in offs:
                ones_v[pl.ds(o, 16)] = jnp.ones((16,), jnp.float32)
        plsc.subcore_barrier()

        base = (c * NS + s) * EPW

        def issue_idx(i, sl):
            pltpu.async_copy(src_hbm.at[pl.ds(base + i * CH, CH)],
                             src_v[sl], semi[sl])
            pltpu.async_copy(dst_hbm.at[pl.ds(base + i * CH, CH)],
                             dst_v[sl], semi[sl])

        def wait_idx(sl):
            for ref in (src_v[sl], dst_v[sl]):
                pltpu.make_async_copy(src_hbm.at[pl.ds(0, CH)], ref,
                                      semi[sl]).wait()

        def issue_gather(sl):
            pltpu.async_copy(x_hbm.at[src_v[sl]], rows_v[sl], semg[sl])

        def wait_gather(sl):
            pltpu.make_async_copy(x_hbm.at[src_v[sl]], rows_v[sl],
                                  semg[sl]).wait()

        def issue_scatter(sl):
            pltpu.async_copy(rows_v[sl], agg_sh.at[dst_v[sl]], sems[sl],
                             add=True)
            if with_deg:
                pltpu.async_copy(ones_v, deg_sh.at[dst_v[sl]], sems[sl],
                                 add=True)

        def wait_scatter(sl):
            pltpu.make_async_copy(rows_v[sl], agg_sh.at[dst_v[sl]],
                                  sems[sl]).wait()
            if with_deg:
                pltpu.make_async_copy(ones_v, deg_sh.at[dst_v[sl]],
                                      sems[sl]).wait()

        def chunk_body(i, j, drain, nxt2, nxt1):
            if drain:
                wait_scatter((j + 2) % K)
            if nxt2:
                issue_idx(i + 2, (j + 2) % K)
            if nxt1:
                wait_idx((j + 1) % K)
                issue_gather((j + 1) % K)
            wait_gather(j)
            issue_scatter(j)

        assert NCHUNK % K == 0 and NCHUNK // K >= 2
        G = NCHUNK // K
        issue_idx(0, 0)
        issue_idx(1, 1)
        wait_idx(0)
        issue_gather(0)
        for j in range(K):
            chunk_body(j, j, j + 2 >= K, True, True)

        def steady(t, _):
            for j in range(K):
                chunk_body(t * K + j, j, True, True, True)
            return _
        lax.fori_loop(1, G - 1, steady, None)

        for j in range(K):
            i = (G - 1) * K + j
            chunk_body(i, j, True, i + 2 < NCHUNK, i + 1 < NCHUNK)
        for sl in (K - 3, K - 2, K - 1):
            wait_scatter(sl)

        plsc.subcore_barrier()
        pltpu.sync_copy(agg_sh.at[pl.ds(s * RPS, RPS)],
                        agg_out.at[c, pl.ds(s * RPS, RPS)])
        if with_deg:
            pltpu.sync_copy(deg_sh.at[pl.ds(s * DPS, DPS)],
                            deg_out.at[c, pl.ds(s * DPS, DPS)])

    params = None
    if d % 128 != 0:
        params = pltpu.CompilerParams(use_tc_tiling_on_sc=False)
    return pl.kernel(body, out_type=tuple(out_type), mesh=mesh,
                     scratch_types=scratch, compiler_params=params)


BN = 2000
GRID = N // BN


def _self1_body(x_ref, ws1_ref, b1_ref, xs_ref):
    xs_ref[...] = x_ref[...] @ ws1_ref[...] + b1_ref[...]


def _dense1_body(xs_ref, p0_ref, p1_ref, deg_ref, wn1_ref,
                 ws2_ref, wn2_ref, y2_ref, z_ref, inv_ref):
    d = jnp.sum(deg_ref[...], axis=0)
    inv = 1.0 / jnp.maximum(d, 1.0)
    inv_ref[...] = inv
    hn = (p0_ref[0] + p1_ref[0]) * inv
    h1 = jnp.maximum(xs_ref[...] + hn @ wn1_ref[...], 0.0)
    y2_ref[...] = h1 @ wn2_ref[...]
    z_ref[...] = h1 @ ws2_ref[...]


def _dense2_body(z_ref, p0_ref, p1_ref, inv_ref, b2_ref, out_ref):
    logits = z_ref[...] + (p0_ref[0] + p1_ref[0]) * inv_ref[...] + b2_ref[...]
    m = jnp.max(logits, axis=1, keepdims=True)
    t = logits - m
    out_ref[...] = t - jnp.log(jnp.sum(jnp.exp(t), axis=1, keepdims=True))


def kernel(x, edge_index, W_self1, W_neigh1, b1, W_self2, W_neigh2, b2):
    src = edge_index[0]
    dst = edge_index[1]

    agg1p, degp = _make_sc_agg(D_IN, True)(x, src, dst)
    degp = degp.reshape(NC, NPAD, 1)

    xs = pl.pallas_call(
        _self1_body,
        grid=(GRID,),
        in_specs=[
            pl.BlockSpec((BN, D_IN), lambda i: (i, 0)),
            pl.BlockSpec((D_IN, D_HID), lambda i: (0, 0)),
            pl.BlockSpec((1, D_HID), lambda i: (0, 0)),
        ],
        out_specs=pl.BlockSpec((BN, D_HID), lambda i: (i, 0)),
        out_shape=jax.ShapeDtypeStruct((N, D_HID), jnp.float32),
    )(x, W_self1, b1.reshape(1, D_HID))

    y2, z, inv_deg = pl.pallas_call(
        _dense1_body,
        grid=(GRID,),
        in_specs=[
            pl.BlockSpec((BN, D_HID), lambda i: (i, 0)),
            pl.BlockSpec((1, BN, D_IN), lambda i: (0, i, 0)),
            pl.BlockSpec((1, BN, D_IN), lambda i: (1, i, 0)),
            pl.BlockSpec((NC, BN, 1), lambda i: (0, i, 0)),
            pl.BlockSpec((D_IN, D_HID), lambda i: (0, 0)),
            pl.BlockSpec((D_HID, D_OUT), lambda i: (0, 0)),
            pl.BlockSpec((D_HID, D_OUT), lambda i: (0, 0)),
        ],
        out_specs=[
            pl.BlockSpec((BN, D_OUT), lambda i: (i, 0)),
            pl.BlockSpec((BN, D_OUT), lambda i: (i, 0)),
            pl.BlockSpec((BN, 1), lambda i: (i, 0)),
        ],
        out_shape=[
            jax.ShapeDtypeStruct((N, D_OUT), jnp.float32),
            jax.ShapeDtypeStruct((N, D_OUT), jnp.float32),
            jax.ShapeDtypeStruct((N, 1), jnp.float32),
        ],
    )(xs, agg1p, agg1p, degp, W_neigh1, W_self2, W_neigh2)

    (agg2p,) = _make_sc_agg(D_OUT, False)(y2, src, dst)

    out = pl.pallas_call(
        _dense2_body,
        grid=(GRID,),
        in_specs=[
            pl.BlockSpec((BN, D_OUT), lambda i: (i, 0)),
            pl.BlockSpec((1, BN, D_OUT), lambda i: (0, i, 0)),
            pl.BlockSpec((1, BN, D_OUT), lambda i: (1, i, 0)),
            pl.BlockSpec((BN, 1), lambda i: (i, 0)),
            pl.BlockSpec((1, D_OUT), lambda i: (0, 0)),
        ],
        out_specs=pl.BlockSpec((BN, D_OUT), lambda i: (i, 0)),
        out_shape=jax.ShapeDtypeStruct((N, D_OUT), jnp.float32),
    )(z, agg2p, agg2p, inv_deg, b2.reshape(1, D_OUT))

    return out


# speedup vs baseline: 13.3550x; 1.0290x over previous
import functools

import jax
import jax.numpy as jnp
from jax import lax
from jax.experimental import pallas as pl
from jax.experimental.pallas import tpu as pltpu
from jax.experimental.pallas import tpu_sc as plsc

N = 10000
E = 320000
D_IN = 128
D_HID = 128
D_OUT = 64

NC = 2
NS = 16
NW = NC * NS
EPW = E // NW
K = 5
NPAD = 10240
RPS = NPAD // NS
DPS = NPAD // NS


@functools.cache
def _make_sc_agg(d, with_deg):
    mesh = plsc.VectorSubcoreMesh(core_axis_name="c", subcore_axis_name="s",
                                  num_cores=NC, num_subcores=NS)
    CH = 40 if d > 64 else 200
    NCHUNK = EPW // CH
    out_type = [jax.ShapeDtypeStruct((NC, NPAD, d), jnp.float32)]
    scratch = (
        [pltpu.VMEM_SHARED((NPAD, d), jnp.float32)]
        + [pltpu.VMEM((CH,), jnp.int32) for _ in range(K)]
        + [pltpu.VMEM((CH,), jnp.int32) for _ in range(K)]
        + [pltpu.VMEM((CH, d), jnp.float32) for _ in range(K)]
        + [pltpu.SemaphoreType.DMA for _ in range(K)]
        + [pltpu.SemaphoreType.DMA for _ in range(K)]
        + [pltpu.SemaphoreType.DMA for _ in range(K)]
    )
    if with_deg:
        out_type.append(jax.ShapeDtypeStruct((NC, NPAD), jnp.float32))
        scratch += [
            pltpu.VMEM_SHARED((NPAD,), jnp.float32),
            pltpu.VMEM((DPS,), jnp.float32),
            pltpu.VMEM((CH,), jnp.float32),
        ]

    def body(x_hbm, src_hbm, dst_hbm, agg_out, *rest):
        rest = list(rest)
        deg_out = rest.pop(0) if with_deg else None
        agg_sh = rest.pop(0)
        src_v = [rest.pop(0) for _ in range(K)]
        dst_v = [rest.pop(0) for _ in range(K)]
        rows_v = [rest.pop(0) for _ in range(K)]
        semi = [rest.pop(0) for _ in range(K)]
        semg = [rest.pop(0) for _ in range(K)]
        sems = [rest.pop(0) for _ in range(K)]
        if with_deg:
            deg_sh, zdeg, ones_v = rest
        c = lax.axis_index("c")
        s = lax.axis_index("s")

        ZR = 40
        def zfill(i, _):
            r = i // (d // 16)
            col = (i % (d // 16)) * 16
            rows_v[0][r, pl.ds(col, 16)] = jnp.zeros((16,), jnp.float32)
            return _
        lax.fori_loop(0, ZR * (d // 16), zfill, None)
        for j in range(RPS // ZR):
            pltpu.sync_copy(rows_v[0].at[pl.ds(0, ZR)],
                            agg_sh.at[pl.ds(s * RPS + j * ZR, ZR)])
        if with_deg:
            def zdfill(i, _):
                zdeg[pl.ds(i * 16, 16)] = jnp.zeros((16,), jnp.float32)
                return _
            lax.fori_loop(0, DPS // 16, zdfill, None)
            pltpu.sync_copy(zdeg, deg_sh.at[pl.ds(s * DPS, DPS)])
            offs = list(range(0, CH - 15, 16))
            if CH % 16:
                offs.append(CH - 16)
            for o in offs:
                ones_v[pl.ds(o, 16)] = jnp.ones((16,), jnp.float32)
        plsc.subcore_barrier()

        base = (c * NS + s) * EPW

        def issue_idx(i, sl):
            pltpu.async_copy(src_hbm.at[pl.ds(base + i * CH, CH)],
                             src_v[sl], semi[sl])
            pltpu.async_copy(dst_hbm.at[pl.ds(base + i * CH, CH)],
                             dst_v[sl], semi[sl])

        def wait_idx(sl):
            for ref in (src_v[sl], dst_v[sl]):
                pltpu.make_async_copy(src_hbm.at[pl.ds(0, CH)], ref,
                                      semi[sl]).wait()

        def issue_gather(sl):
            pltpu.async_copy(x_hbm.at[src_v[sl]], rows_v[sl], semg[sl])

        def wait_gather(sl):
            pltpu.make_async_copy(x_hbm.at[src_v[sl]], rows_v[sl],
                                  semg[sl]).wait()

        def issue_scatter(sl):
            pltpu.async_copy(rows_v[sl], agg_sh.at[dst_v[sl]], sems[sl],
                             add=True)
            if with_deg:
                pltpu.async_copy(ones_v, deg_sh.at[dst_v[sl]], sems[sl],
                                 add=True)

        def wait_scatter(sl):
            pltpu.make_async_copy(rows_v[sl], agg_sh.at[dst_v[sl]],
                                  sems[sl]).wait()
            if with_deg:
                pltpu.make_async_copy(ones_v, deg_sh.at[dst_v[sl]],
                                      sems[sl]).wait()

        def chunk_body(i, j, drain, nxt2, nxt1):
            if drain:
                wait_scatter((j + 2) % K)
            if nxt2:
                issue_idx(i + 2, (j + 2) % K)
            if nxt1:
                wait_idx((j + 1) % K)
                issue_gather((j + 1) % K)
            wait_gather(j)
            issue_scatter(j)

        assert NCHUNK % K == 0 and NCHUNK // K >= 2
        G = NCHUNK // K
        issue_idx(0, 0)
        issue_idx(1, 1)
        wait_idx(0)
        issue_gather(0)
        for j in range(K):
            chunk_body(j, j, j + 2 >= K, True, True)

        def steady(t, _):
            for j in range(K):
                chunk_body(t * K + j, j, True, True, True)
            return _
        lax.fori_loop(1, G - 1, steady, None)

        for j in range(K):
            i = (G - 1) * K + j
            chunk_body(i, j, True, i + 2 < NCHUNK, i + 1 < NCHUNK)
        for sl in (K - 3, K - 2, K - 1):
            wait_scatter(sl)

        plsc.subcore_barrier()
        pltpu.sync_copy(agg_sh.at[pl.ds(s * RPS, RPS)],
                        agg_out.at[c, pl.ds(s * RPS, RPS)])
        if with_deg:
            pltpu.sync_copy(deg_sh.at[pl.ds(s * DPS, DPS)],
                            deg_out.at[c, pl.ds(s * DPS, DPS)])

    params = None
    if d % 128 != 0:
        params = pltpu.CompilerParams(use_tc_tiling_on_sc=False)
    return pl.kernel(body, out_type=tuple(out_type), mesh=mesh,
                     scratch_types=scratch, compiler_params=params)


EPS = E // NS


@functools.cache
def _make_sc_agg_split(d):
    mesh = plsc.VectorSubcoreMesh(core_axis_name="c", subcore_axis_name="s",
                                  num_cores=NC, num_subcores=NS)
    CH = 200
    NCHUNK = EPS // CH
    out_type = (jax.ShapeDtypeStruct((NC, NPAD, d), jnp.float32),
                jax.ShapeDtypeStruct((NC, NPAD), jnp.float32))
    scratch = (
        [pltpu.VMEM_SHARED((NPAD, d), jnp.float32)]
        + [pltpu.VMEM((CH,), jnp.int32) for _ in range(K)]
        + [pltpu.VMEM((CH,), jnp.int32) for _ in range(K)]
        + [pltpu.VMEM((CH, d), jnp.float32) for _ in range(K)]
        + [pltpu.SemaphoreType.DMA for _ in range(K)]
        + [pltpu.SemaphoreType.DMA for _ in range(K)]
        + [pltpu.SemaphoreType.DMA for _ in range(K)]
        + [pltpu.VMEM_SHARED((NPAD,), jnp.float32),
           pltpu.VMEM((DPS,), jnp.float32),
           pltpu.VMEM((CH,), jnp.float32)]
    )

    def body(x_hbm, src_hbm, dst_hbm, agg_out, deg_out, agg_sh, *rest):
        rest = list(rest)
        src_v = [rest.pop(0) for _ in range(K)]
        dst_v = [rest.pop(0) for _ in range(K)]
        rows_v = [rest.pop(0) for _ in range(K)]
        semi = [rest.pop(0) for _ in range(K)]
        semg = [rest.pop(0) for _ in range(K)]
        sems = [rest.pop(0) for _ in range(K)]
        deg_sh, zdeg, ones_v = rest
        c = lax.axis_index("c")
        s = lax.axis_index("s")

        ZR = 40
        def zfill(i, _):
            r = i // (d // 16)
            col = (i % (d // 16)) * 16
            rows_v[0][r, pl.ds(col, 16)] = jnp.zeros((16,), jnp.float32)
            return _
        lax.fori_loop(0, ZR * (d // 16), zfill, None)
        for j in range(RPS // ZR):
            pltpu.sync_copy(rows_v[0].at[pl.ds(0, ZR)],
                            agg_sh.at[pl.ds(s * RPS + j * ZR, ZR)])
        def zdfill(i, _):
            zdeg[pl.ds(i * 16, 16)] = jnp.zeros((16,), jnp.float32)
            return _
        lax.fori_loop(0, DPS // 16, zdfill, None)
        pltpu.sync_copy(zdeg, deg_sh.at[pl.ds(s * DPS, DPS)])
        offs = list(range(0, CH - 15, 16))
        if CH % 16:
            offs.append(CH - 16)
        for o in offs:
            ones_v[pl.ds(o, 16)] = jnp.ones((16,), jnp.float32)
        plsc.subcore_barrier()

        base = s * EPS

        def issue_idx(i, sl):
            pltpu.async_copy(src_hbm.at[pl.ds(base + i * CH, CH)],
                             src_v[sl], semi[sl])
            pltpu.async_copy(dst_hbm.at[pl.ds(base + i * CH, CH)],
                             dst_v[sl], semi[sl])

        def wait_idx(sl):
            for ref in (src_v[sl], dst_v[sl]):
                pltpu.make_async_copy(src_hbm.at[pl.ds(0, CH)], ref,
                                      semi[sl]).wait()

        def issue_gather(sl):
            pltpu.async_copy(x_hbm.at[c].at[src_v[sl]], rows_v[sl], semg[sl])

        def wait_gather(sl):
            pltpu.make_async_copy(x_hbm.at[c].at[src_v[sl]], rows_v[sl],
                                  semg[sl]).wait()

        def issue_scatter(sl):
            pltpu.async_copy(rows_v[sl], agg_sh.at[dst_v[sl]], sems[sl],
                             add=True)
            pltpu.async_copy(ones_v, deg_sh.at[dst_v[sl]], sems[sl],
                             add=True)

        def wait_scatter(sl):
            pltpu.make_async_copy(rows_v[sl], agg_sh.at[dst_v[sl]],
                                  sems[sl]).wait()
            pltpu.make_async_copy(ones_v, deg_sh.at[dst_v[sl]],
                                  sems[sl]).wait()

        def chunk_body(i, j, drain, nxt2, nxt1):
            if drain:
                wait_scatter((j + 2) % K)
            if nxt2:
                issue_idx(i + 2, (j + 2) % K)
            if nxt1:
                wait_idx((j + 1) % K)
                issue_gather((j + 1) % K)
            wait_gather(j)
            issue_scatter(j)

        assert NCHUNK % K == 0 and NCHUNK // K >= 2
        G = NCHUNK // K
        issue_idx(0, 0)
        issue_idx(1, 1)
        wait_idx(0)
        issue_gather(0)
        for j in range(K):
            chunk_body(j, j, j + 2 >= K, True, True)

        def steady(t, _):
            for j in range(K):
                chunk_body(t * K + j, j, True, True, True)
            return _
        lax.fori_loop(1, G - 1, steady, None)

        for j in range(K):
            i = (G - 1) * K + j
            chunk_body(i, j, True, i + 2 < NCHUNK, i + 1 < NCHUNK)
        for sl in (K - 3, K - 2, K - 1):
            wait_scatter(sl)

        plsc.subcore_barrier()
        pltpu.sync_copy(agg_sh.at[pl.ds(s * RPS, RPS)],
                        agg_out.at[c, pl.ds(s * RPS, RPS)])
        pltpu.sync_copy(deg_sh.at[pl.ds(s * DPS, DPS)],
                        deg_out.at[c, pl.ds(s * DPS, DPS)])

    params = pltpu.CompilerParams(use_tc_tiling_on_sc=False)
    return pl.kernel(body, out_type=out_type, mesh=mesh,
                     scratch_types=scratch, compiler_params=params)


BN = 2000
GRID = N // BN


def _self1_body(x_ref, ws1_ref, b1_ref, xs_ref):
    xs_ref[...] = x_ref[...] @ ws1_ref[...] + b1_ref[...]


def _dense1_body(xs_ref, p0_ref, p1_ref, deg_ref, wn1_ref,
                 ws2_ref, wn2_ref, y2_ref, z_ref, inv_ref):
    inv = 1.0 / jnp.maximum(deg_ref[0], 1.0)
    inv_ref[...] = inv
    hn = jnp.concatenate([p0_ref[0], p1_ref[0]], axis=1) * inv
    h1 = jnp.maximum(xs_ref[...] + hn @ wn1_ref[...], 0.0)
    y2_ref[...] = h1 @ wn2_ref[...]
    z_ref[...] = h1 @ ws2_ref[...]


def _dense2_body(z_ref, p0_ref, p1_ref, inv_ref, b2_ref, out_ref):
    logits = z_ref[...] + (p0_ref[0] + p1_ref[0]) * inv_ref[...] + b2_ref[...]
    m = jnp.max(logits, axis=1, keepdims=True)
    t = logits - m
    out_ref[...] = t - jnp.log(jnp.sum(jnp.exp(t), axis=1, keepdims=True))


def kernel(x, edge_index, W_self1, W_neigh1, b1, W_self2, W_neigh2, b2):
    src = edge_index[0]
    dst = edge_index[1]

    xsplit = x.reshape(N, NC, D_IN // NC).transpose(1, 0, 2)
    agg1p, degp = _make_sc_agg_split(D_IN // NC)(xsplit, src, dst)
    degp = degp.reshape(NC, NPAD, 1)

    xs = pl.pallas_call(
        _self1_body,
        grid=(GRID,),
        in_specs=[
            pl.BlockSpec((BN, D_IN), lambda i: (i, 0)),
            pl.BlockSpec((D_IN, D_HID), lambda i: (0, 0)),
            pl.BlockSpec((1, D_HID), lambda i: (0, 0)),
        ],
        out_specs=pl.BlockSpec((BN, D_HID), lambda i: (i, 0)),
        out_shape=jax.ShapeDtypeStruct((N, D_HID), jnp.float32),
    )(x, W_self1, b1.reshape(1, D_HID))

    y2, z, inv_deg = pl.pallas_call(
        _dense1_body,
        grid=(GRID,),
        in_specs=[
            pl.BlockSpec((BN, D_HID), lambda i: (i, 0)),
            pl.BlockSpec((1, BN, D_IN // NC), lambda i: (0, i, 0)),
            pl.BlockSpec((1, BN, D_IN // NC), lambda i: (1, i, 0)),
            pl.BlockSpec((1, BN, 1), lambda i: (0, i, 0)),
            pl.BlockSpec((D_IN, D_HID), lambda i: (0, 0)),
            pl.BlockSpec((D_HID, D_OUT), lambda i: (0, 0)),
            pl.BlockSpec((D_HID, D_OUT), lambda i: (0, 0)),
        ],
        out_specs=[
            pl.BlockSpec((BN, D_OUT), lambda i: (i, 0)),
            pl.BlockSpec((BN, D_OUT), lambda i: (i, 0)),
            pl.BlockSpec((BN, 1), lambda i: (i, 0)),
        ],
        out_shape=[
            jax.ShapeDtypeStruct((N, D_OUT), jnp.float32),
            jax.ShapeDtypeStruct((N, D_OUT), jnp.float32),
            jax.ShapeDtypeStruct((N, 1), jnp.float32),
        ],
    )(xs, agg1p, agg1p, degp, W_neigh1, W_self2, W_neigh2)

    (agg2p,) = _make_sc_agg(D_OUT, False)(y2, src, dst)

    out = pl.pallas_call(
        _dense2_body,
        grid=(GRID,),
        in_specs=[
            pl.BlockSpec((BN, D_OUT), lambda i: (i, 0)),
            pl.BlockSpec((1, BN, D_OUT), lambda i: (0, i, 0)),
            pl.BlockSpec((1, BN, D_OUT), lambda i: (1, i, 0)),
            pl.BlockSpec((BN, 1), lambda i: (i, 0)),
            pl.BlockSpec((1, D_OUT), lambda i: (0, 0)),
        ],
        out_specs=pl.BlockSpec((BN, D_OUT), lambda i: (i, 0)),
        out_shape=jax.ShapeDtypeStruct((N, D_OUT), jnp.float32),
    )(z, agg2p, agg2p, inv_deg, b2.reshape(1, D_OUT))

    return out
